# baseline (speedup 1.0000x reference)
import jax
import jax.numpy as jnp
from jax.experimental import pallas as pl
from jax.experimental.pallas import tpu as pltpu

_B, _S, _D = 4, 4096, 2048
_T = 3
_ROWS = 512


def _concat_kernel(x_ref, emb_ref, out_ref, carry_ref):
    j = pl.program_id(1)
    nj = pl.num_programs(1)

    @pl.when(j == 0)
    def _():
        out_ref[0, 0:_T] = emb_ref[...]

    @pl.when(j > 0)
    def _():
        out_ref[0, 0:_T] = carry_ref[0:_T]

    @pl.when(j < nj - 1)
    def _():
        out_ref[0, _T:_ROWS] = x_ref[0, 0:_ROWS - _T]
        carry_ref[0:_T] = x_ref[0, _ROWS - _T:_ROWS]


def kernel(x, embedding):
    nxb = _S // _ROWS
    grid = (_B, nxb + 1)
    return pl.pallas_call(
        _concat_kernel,
        grid=grid,
        out_shape=jax.ShapeDtypeStruct((_B, _S + _T, _D), x.dtype),
        in_specs=[
            pl.BlockSpec((1, _ROWS, _D),
                         lambda b, j: (b, jnp.minimum(j, nxb - 1), 0)),
            pl.BlockSpec((_T, _D), lambda b, j: (0, 0)),
        ],
        out_specs=pl.BlockSpec((1, _ROWS, _D), lambda b, j: (b, j, 0)),
        scratch_shapes=[pltpu.VMEM((8, _D), x.dtype)],
    )(x, embedding)


# speedup vs baseline: 1.0176x; 1.0176x over previous
import jax
import jax.numpy as jnp
from jax.experimental import pallas as pl
from jax.experimental.pallas import tpu as pltpu

_B, _S, _D = 4, 4096, 2048
_T = 3
_ROWS = 1024


def _concat_kernel(x_ref, emb_ref, out_ref, carry_ref):
    j = pl.program_id(1)
    nj = pl.num_programs(1)

    @pl.when(j == 0)
    def _():
        out_ref[0, 0:_T] = emb_ref[...]

    @pl.when(j > 0)
    def _():
        out_ref[0, 0:_T] = carry_ref[0:_T]

    @pl.when(j < nj - 1)
    def _():
        out_ref[0, _T:_ROWS] = x_ref[0, 0:_ROWS - _T]
        carry_ref[0:_T] = x_ref[0, _ROWS - _T:_ROWS]


def kernel(x, embedding):
    nxb = _S // _ROWS
    grid = (_B, nxb + 1)
    return pl.pallas_call(
        _concat_kernel,
        grid=grid,
        out_shape=jax.ShapeDtypeStruct((_B, _S + _T, _D), x.dtype),
        in_specs=[
            pl.BlockSpec((1, _ROWS, _D),
                         lambda b, j: (b, jnp.minimum(j, nxb - 1), 0)),
            pl.BlockSpec((_T, _D), lambda b, j: (0, 0)),
        ],
        out_specs=pl.BlockSpec((1, _ROWS, _D), lambda b, j: (b, j, 0)),
        scratch_shapes=[pltpu.VMEM((8, _D), x.dtype)],
    )(x, embedding)
